# baseline (speedup 1.0000x reference)
import functools

import jax
import jax.numpy as jnp
import numpy as np
from jax.experimental import pallas as pl
from jax.experimental.pallas import tpu as pltpu

_B = 16384
_OBS = 256
_OUT = 64
_H = 512
_K = 10
_KP = 16
_TEST = 100
_TP = 128
_BT = 1024
_NT = _B // _BT
_LOG2PI = float(np.log(2.0 * np.pi))


def _memo_body(x_ref, dx_ref, a_ref,
               ve_w1, ve_b1, ve_w2, ve_b2, pre_w, pre_b,
               embT, emb, emb_sq, post_w, post_b,
               vd_w1, vd_b1, vd_w2, vd_b2,
               d_w1a, d_w1b, d_b1, d_w2, d_b2, d_w3, d_b3, d_w4, d_b4,
               ls_ref,
               loss_ref, reconl_ref, prop_ref, vqt_ref, sums_ref):
    i = pl.program_id(0)

    @pl.when(i < _NT)
    def _compute():
        dx = dx_ref[...]
        h = jnp.tanh(jnp.dot(dx, ve_w1[...],
                             preferred_element_type=jnp.float32) + ve_b1[...])
        enc0 = jnp.dot(h, ve_w2[...],
                       preferred_element_type=jnp.float32) + ve_b2[...]
        enc = jnp.dot(enc0, pre_w[...],
                      preferred_element_type=jnp.float32) + pre_b[...]
        score = emb_sq[...] - 2.0 * jnp.dot(enc, embT[...],
                                            preferred_element_type=jnp.float32)
        mind = jnp.min(score, axis=1, keepdims=True)
        idxr = jax.lax.broadcasted_iota(jnp.int32, (_BT, _KP), 1)
        prop = jnp.min(jnp.where(score == mind, idxr, _KP), axis=1)
        onehot = (idxr == prop[:, None]).astype(jnp.float32)
        quant = jnp.dot(onehot, emb[...], preferred_element_type=jnp.float32)
        b16 = jnp.bfloat16
        postq = jnp.dot(quant.astype(b16), post_w[...],
                        preferred_element_type=jnp.float32) + post_b[...]
        t1 = jnp.tanh(jnp.dot(postq.astype(b16), vd_w1[...],
                              preferred_element_type=jnp.float32) + vd_b1[...])
        recon = jnp.tanh(jnp.dot(t1.astype(b16), vd_w2[...],
                                 preferred_element_type=jnp.float32) + vd_b2[...])
        propf = prop.astype(jnp.float32)
        h1 = jax.nn.relu(jnp.dot(x_ref[...], d_w1a[...],
                                 preferred_element_type=jnp.float32)
                         + propf[:, None] * d_w1b[...] + d_b1[...])
        h2 = jax.nn.relu(jnp.dot(h1.astype(b16), d_w2[...],
                                 preferred_element_type=jnp.float32) + d_b2[...])
        h3 = jnp.tanh(jax.nn.relu(jnp.dot(h2.astype(b16), d_w3[...],
                                          preferred_element_type=jnp.float32)
                                  + d_b3[...]))
        mu = jnp.dot(h3.astype(b16), d_w4[...],
                     preferred_element_type=jnp.float32) + d_b4[...]
        ls = ls_ref[...]
        z = (a_ref[...] - mu) * jnp.exp(-ls)
        rl = jnp.sum(0.5 * z * z + ls + 0.5 * _LOG2PI, axis=1)
        reconl_ref[i, :] = rl
        prop_ref[i, :] = prop
        dr = dx - recon
        dq = enc - quant
        s_r = jnp.sum(dr * dr)
        s_q = jnp.sum(dq * dq)
        prev_r = jnp.where(i == 0, 0.0, sums_ref[0])
        prev_q = jnp.where(i == 0, 0.0, sums_ref[1])
        sums_ref[0] = prev_r + s_r
        sums_ref[1] = prev_q + s_q

    @pl.when(i == _NT)
    def _finalize():
        vq_total = (sums_ref[0] + 2.0 * sums_ref[1]) * (1.0 / (_B * _OBS))
        vqt_ref[...] = jnp.full((1, 128), vq_total, jnp.float32)
        loss_ref[...] = reconl_ref[...] * vq_total


def _tile_map(i):
    return (jnp.minimum(i, _NT - 1), 0)


def _whole(i):
    return (0, 0)


def kernel(X, Delta_X, A, context_sample, con_dim, ve_w1, ve_b1, ve_w2, ve_b2,
           pre_w, pre_b, emb, post_w, post_b, vd_w1, vd_b1, vd_w2, vd_b2,
           d_w1, d_b1, d_w2, d_b2, d_w3, d_b3, d_w4, d_b4, log_std):
    b16 = jnp.bfloat16
    emb_p = jnp.zeros((_KP, _OBS), jnp.float32).at[:_K].set(emb)
    embT = emb_p.T
    emb_sq = jnp.full((1, _KP), 1e30, jnp.float32).at[0, :_K].set(
        jnp.sum(emb * emb, axis=1))
    d_w1a = d_w1[:_OBS].astype(b16)
    d_w1b = d_w1[_OBS:]
    vd_w1p = jnp.zeros((_H, _TP), b16).at[:, :_TEST].set(vd_w1.astype(b16))
    vd_b1p = jnp.zeros((1, _TP), jnp.float32).at[0, :_TEST].set(vd_b1)
    vd_w2p = jnp.zeros((_TP, _OBS), b16).at[:_TEST].set(vd_w2.astype(b16))

    def row(v):
        return v.reshape(1, -1)

    ins = (X.astype(b16), Delta_X, A,
           ve_w1, row(ve_b1), ve_w2, row(ve_b2), pre_w, row(pre_b),
           embT, emb_p, emb_sq, post_w.astype(b16), row(post_b),
           vd_w1p, vd_b1p, vd_w2p, row(vd_b2),
           d_w1a, d_w1b, row(d_b1), d_w2.astype(b16), row(d_b2),
           d_w3.astype(b16), row(d_b3), d_w4.astype(b16), row(d_b4),
           row(log_std))

    in_specs = [
        pl.BlockSpec((_BT, _OBS), _tile_map),
        pl.BlockSpec((_BT, _OBS), _tile_map),
        pl.BlockSpec((_BT, _OUT), _tile_map),
    ] + [pl.BlockSpec(v.shape, _whole) for v in ins[3:]]

    out_shapes = (
        jax.ShapeDtypeStruct((_NT, _BT), jnp.float32),
        jax.ShapeDtypeStruct((_NT, _BT), jnp.float32),
        jax.ShapeDtypeStruct((_NT, _BT), jnp.int32),
        jax.ShapeDtypeStruct((1, 128), jnp.float32),
    )
    out_specs = (
        pl.BlockSpec((_NT, _BT), _whole),
        pl.BlockSpec((_NT, _BT), _whole),
        pl.BlockSpec((_NT, _BT), _whole),
        pl.BlockSpec((1, 128), _whole),
    )

    loss2, recon2, prop2, vqt = pl.pallas_call(
        _memo_body,
        grid=(_NT + 1,),
        in_specs=in_specs,
        out_specs=out_specs,
        out_shape=out_shapes,
        scratch_shapes=[pltpu.SMEM((2,), jnp.float32)],
        compiler_params=pltpu.CompilerParams(
            dimension_semantics=("arbitrary",)),
    )(*ins)

    return (loss2.reshape(_B), recon2.reshape(_B), X, prop2.reshape(_B),
            vqt[0, 0])


# speedup vs baseline: 1.1030x; 1.1030x over previous
import functools

import jax
import jax.numpy as jnp
import numpy as np
from jax.experimental import pallas as pl
from jax.experimental.pallas import tpu as pltpu

_B = 16384
_OBS = 256
_OUT = 64
_H = 512
_K = 10
_KP = 16
_TEST = 100
_TP = 128
_BT = 1024
_NT = _B // _BT
_LOG2PI = float(np.log(2.0 * np.pi))


def _memo_body(x_ref, dx_ref, a_ref,
               ve_w1, ve_b1, ve_w2, ve_b2, pre_w, pre_b,
               embT, emb, emb_sq, post_w, post_b,
               vd_w1, vd_b1, vd_w2, vd_b2,
               d_w1a, d_w1b, d_b1, d_w2, d_b2, d_w3, d_b3, d_w4, d_b4,
               ls_ref,
               loss_ref, reconl_ref, prop_ref, vqt_ref, sums_ref):
    i = pl.program_id(0)

    @pl.when(i < _NT)
    def _compute():
        dx = dx_ref[...]
        h = jnp.tanh(jnp.dot(dx, ve_w1[...],
                             preferred_element_type=jnp.float32) + ve_b1[...])
        enc0 = jnp.dot(h, ve_w2[...],
                       preferred_element_type=jnp.float32) + ve_b2[...]
        enc = jnp.dot(enc0, pre_w[...],
                      preferred_element_type=jnp.float32) + pre_b[...]
        score = emb_sq[...] - 2.0 * jnp.dot(enc, embT[...],
                                            preferred_element_type=jnp.float32)
        mind = jnp.min(score, axis=1, keepdims=True)
        idxr = jax.lax.broadcasted_iota(jnp.int32, (_BT, _KP), 1)
        prop = jnp.min(jnp.where(score == mind, idxr, _KP), axis=1)
        onehot = (idxr == prop[:, None]).astype(jnp.float32)
        quant = jnp.dot(onehot, emb[...], preferred_element_type=jnp.float32)
        postq = jnp.dot(quant, post_w[...],
                        preferred_element_type=jnp.float32) + post_b[...]
        t1 = jnp.tanh(jnp.dot(postq, vd_w1[...],
                              preferred_element_type=jnp.float32) + vd_b1[...])
        recon = jnp.tanh(jnp.dot(t1, vd_w2[...],
                                 preferred_element_type=jnp.float32) + vd_b2[...])
        propf = prop.astype(jnp.float32)
        h1 = jax.nn.relu(jnp.dot(x_ref[...], d_w1a[...],
                                 preferred_element_type=jnp.float32)
                         + propf[:, None] * d_w1b[...] + d_b1[...])
        h2 = jax.nn.relu(jnp.dot(h1, d_w2[...],
                                 preferred_element_type=jnp.float32) + d_b2[...])
        h3 = jnp.tanh(jax.nn.relu(jnp.dot(h2, d_w3[...],
                                          preferred_element_type=jnp.float32)
                                  + d_b3[...]))
        mu = jnp.dot(h3, d_w4[...],
                     preferred_element_type=jnp.float32) + d_b4[...]
        ls = ls_ref[...]
        z = (a_ref[...] - mu) * jnp.exp(-ls)
        rl = jnp.sum(0.5 * z * z + ls + 0.5 * _LOG2PI, axis=1)
        reconl_ref[i, :] = rl
        prop_ref[i, :] = prop
        dr = dx - recon
        dq = enc - quant
        s_r = jnp.sum(dr * dr)
        s_q = jnp.sum(dq * dq)
        prev_r = jnp.where(i == 0, 0.0, sums_ref[0])
        prev_q = jnp.where(i == 0, 0.0, sums_ref[1])
        sums_ref[0] = prev_r + s_r
        sums_ref[1] = prev_q + s_q

    @pl.when(i == _NT)
    def _finalize():
        vq_total = (sums_ref[0] + 2.0 * sums_ref[1]) * (1.0 / (_B * _OBS))
        vqt_ref[...] = jnp.full((1, 128), vq_total, jnp.float32)
        loss_ref[...] = reconl_ref[...] * vq_total


def _tile_map(i):
    return (jnp.minimum(i, _NT - 1), 0)


def _whole(i):
    return (0, 0)


def kernel(X, Delta_X, A, context_sample, con_dim, ve_w1, ve_b1, ve_w2, ve_b2,
           pre_w, pre_b, emb, post_w, post_b, vd_w1, vd_b1, vd_w2, vd_b2,
           d_w1, d_b1, d_w2, d_b2, d_w3, d_b3, d_w4, d_b4, log_std):
    emb_p = jnp.zeros((_KP, _OBS), jnp.float32).at[:_K].set(emb)
    embT = emb_p.T
    emb_sq = jnp.full((1, _KP), 1e30, jnp.float32).at[0, :_K].set(
        jnp.sum(emb * emb, axis=1))
    d_w1a = d_w1[:_OBS]
    d_w1b = d_w1[_OBS:]
    vd_w1p = jnp.zeros((_H, _TP), jnp.float32).at[:, :_TEST].set(vd_w1)
    vd_b1p = jnp.zeros((1, _TP), jnp.float32).at[0, :_TEST].set(vd_b1)
    vd_w2p = jnp.zeros((_TP, _OBS), jnp.float32).at[:_TEST].set(vd_w2)

    def row(v):
        return v.reshape(1, -1)

    ins = (X, Delta_X, A,
           ve_w1, row(ve_b1), ve_w2, row(ve_b2), pre_w, row(pre_b),
           embT, emb_p, emb_sq, post_w, row(post_b),
           vd_w1p, vd_b1p, vd_w2p, row(vd_b2),
           d_w1a, d_w1b, row(d_b1), d_w2, row(d_b2), d_w3, row(d_b3),
           d_w4, row(d_b4), row(log_std))

    in_specs = [
        pl.BlockSpec((_BT, _OBS), _tile_map),
        pl.BlockSpec((_BT, _OBS), _tile_map),
        pl.BlockSpec((_BT, _OUT), _tile_map),
    ] + [pl.BlockSpec(v.shape, _whole) for v in ins[3:]]

    out_shapes = (
        jax.ShapeDtypeStruct((_NT, _BT), jnp.float32),
        jax.ShapeDtypeStruct((_NT, _BT), jnp.float32),
        jax.ShapeDtypeStruct((_NT, _BT), jnp.int32),
        jax.ShapeDtypeStruct((1, 128), jnp.float32),
    )
    out_specs = (
        pl.BlockSpec((_NT, _BT), _whole),
        pl.BlockSpec((_NT, _BT), _whole),
        pl.BlockSpec((_NT, _BT), _whole),
        pl.BlockSpec((1, 128), _whole),
    )

    loss2, recon2, prop2, vqt = pl.pallas_call(
        _memo_body,
        grid=(_NT + 1,),
        in_specs=in_specs,
        out_specs=out_specs,
        out_shape=out_shapes,
        scratch_shapes=[pltpu.SMEM((2,), jnp.float32)],
        compiler_params=pltpu.CompilerParams(
            dimension_semantics=("arbitrary",)),
    )(*ins)

    return (loss2.reshape(_B), recon2.reshape(_B), X, prop2.reshape(_B),
            vqt[0, 0])


# speedup vs baseline: 1.1713x; 1.0619x over previous
import jax
import jax.numpy as jnp
import numpy as np
from jax.experimental import pallas as pl
from jax.experimental.pallas import tpu as pltpu

_B = 16384
_OBS = 256
_OUT = 64
_H = 512
_K = 10
_TEST = 100
_BT = 1024
_NT = _B // _BT
_LOG2PI = float(np.log(2.0 * np.pi))


def _memo_body(x_ref, dx_ref, a_ref,
               ve_w1, ve_b1, ve_w2, ve_b2, pre_w, pre_b,
               embT, emb, emb_sq, post_w, post_b,
               vd_w1, vd_b1, vd_w2, vd_b2,
               d_w1a, d_w1b, d_b1, d_w2, d_b2, d_w3, d_b3, d_w4, d_b4,
               ls_ref,
               reconl_ref, prop_ref, sr_ref, sq_ref):
    dx = dx_ref[...]
    h = jnp.tanh(jnp.dot(dx, ve_w1[...],
                         preferred_element_type=jnp.float32) + ve_b1[...])
    enc0 = jnp.dot(h, ve_w2[...],
                   preferred_element_type=jnp.float32) + ve_b2[...]
    enc = jnp.dot(enc0, pre_w[...],
                  preferred_element_type=jnp.float32) + pre_b[...]
    score = emb_sq[...] - 2.0 * jnp.dot(enc, embT[...],
                                        preferred_element_type=jnp.float32)
    mind = jnp.min(score, axis=1, keepdims=True)
    idxr = jax.lax.broadcasted_iota(jnp.int32, (_BT, _K), 1)
    prop = jnp.min(jnp.where(score == mind, idxr, _K), axis=1)
    onehot = (idxr == prop[:, None]).astype(jnp.float32)
    quant = jnp.dot(onehot, emb[...], preferred_element_type=jnp.float32)
    postq = jnp.dot(quant, post_w[...],
                    preferred_element_type=jnp.float32) + post_b[...]
    t1 = jnp.tanh(jnp.dot(postq, vd_w1[...],
                          preferred_element_type=jnp.float32) + vd_b1[...])
    recon = jnp.tanh(jnp.dot(t1, vd_w2[...],
                             preferred_element_type=jnp.float32) + vd_b2[...])
    propf = prop.astype(jnp.float32)
    h1 = jax.nn.relu(jnp.dot(x_ref[...], d_w1a[...],
                             preferred_element_type=jnp.float32)
                     + propf[:, None] * d_w1b[...] + d_b1[...])
    h2 = jax.nn.relu(jnp.dot(h1, d_w2[...],
                             preferred_element_type=jnp.float32) + d_b2[...])
    h3 = jnp.tanh(jax.nn.relu(jnp.dot(h2, d_w3[...],
                                      preferred_element_type=jnp.float32)
                              + d_b3[...]))
    mu = jnp.dot(h3, d_w4[...],
                 preferred_element_type=jnp.float32) + d_b4[...]
    ls = ls_ref[...]
    z = (a_ref[...] - mu) * jnp.exp(-ls)
    rl = jnp.sum(0.5 * z * z + ls + 0.5 * _LOG2PI, axis=1)
    reconl_ref[0, 0, :] = rl
    prop_ref[0, 0, :] = prop
    dr = dx - recon
    dq = enc - quant
    sr_ref[...] = jnp.full((1, 1, 128), jnp.sum(dr * dr), jnp.float32)
    sq_ref[...] = jnp.full((1, 1, 128), jnp.sum(dq * dq), jnp.float32)


def _final_body(reconl_ref, sr_ref, sq_ref, loss_ref, vqt_ref):
    tot = (jnp.sum(sr_ref[...]) + 2.0 * jnp.sum(sq_ref[...])) / 128.0
    vq_total = tot * (1.0 / (_B * _OBS))
    vqt_ref[...] = jnp.full((1, 128), vq_total, jnp.float32)
    loss_ref[...] = reconl_ref[...].reshape(_NT, _BT) * vq_total


def _tile_map(i):
    return (i, 0)


def _whole(i):
    return (0, 0)


def kernel(X, Delta_X, A, context_sample, con_dim, ve_w1, ve_b1, ve_w2, ve_b2,
           pre_w, pre_b, emb, post_w, post_b, vd_w1, vd_b1, vd_w2, vd_b2,
           d_w1, d_b1, d_w2, d_b2, d_w3, d_b3, d_w4, d_b4, log_std):
    embT = emb.T
    emb_sq = jnp.sum(emb * emb, axis=1)[None, :]
    d_w1a = d_w1[:_OBS]
    d_w1b = d_w1[_OBS:]

    def row(v):
        return v.reshape(1, -1)

    ins = (X, Delta_X, A,
           ve_w1, row(ve_b1), ve_w2, row(ve_b2), pre_w, row(pre_b),
           embT, emb, emb_sq, post_w, row(post_b),
           vd_w1, row(vd_b1), vd_w2, row(vd_b2),
           d_w1a, d_w1b, row(d_b1), d_w2, row(d_b2), d_w3, row(d_b3),
           d_w4, row(d_b4), row(log_std))

    in_specs = [
        pl.BlockSpec((_BT, _OBS), _tile_map),
        pl.BlockSpec((_BT, _OBS), _tile_map),
        pl.BlockSpec((_BT, _OUT), _tile_map),
    ] + [pl.BlockSpec(v.shape, _whole) for v in ins[3:]]

    recon2, prop3, sr, sq = pl.pallas_call(
        _memo_body,
        grid=(_NT,),
        in_specs=in_specs,
        out_specs=(
            pl.BlockSpec((1, 1, _BT), lambda i: (i, 0, 0)),
            pl.BlockSpec((1, 1, _BT), lambda i: (i, 0, 0)),
            pl.BlockSpec((1, 1, 128), lambda i: (i, 0, 0)),
            pl.BlockSpec((1, 1, 128), lambda i: (i, 0, 0)),
        ),
        out_shape=(
            jax.ShapeDtypeStruct((_NT, 1, _BT), jnp.float32),
            jax.ShapeDtypeStruct((_NT, 1, _BT), jnp.int32),
            jax.ShapeDtypeStruct((_NT, 1, 128), jnp.float32),
            jax.ShapeDtypeStruct((_NT, 1, 128), jnp.float32),
        ),
        compiler_params=pltpu.CompilerParams(
            dimension_semantics=("parallel",)),
    )(*ins)

    loss2, vqt = pl.pallas_call(
        _final_body,
        in_specs=[
            pl.BlockSpec((_NT, 1, _BT), lambda: (0, 0, 0)),
            pl.BlockSpec((_NT, 1, 128), lambda: (0, 0, 0)),
            pl.BlockSpec((_NT, 1, 128), lambda: (0, 0, 0)),
        ],
        out_specs=(
            pl.BlockSpec((_NT, _BT), lambda: (0, 0)),
            pl.BlockSpec((1, 128), lambda: (0, 0)),
        ),
        out_shape=(
            jax.ShapeDtypeStruct((_NT, _BT), jnp.float32),
            jax.ShapeDtypeStruct((1, 128), jnp.float32),
        ),
    )(recon2, sr, sq)

    return (loss2.reshape(_B), recon2.reshape(_B), X, prop3.reshape(_B),
            vqt[0, 0])


# speedup vs baseline: 1.2027x; 1.0268x over previous
import jax
import jax.numpy as jnp
import numpy as np
from jax.experimental import pallas as pl
from jax.experimental.pallas import tpu as pltpu

_B = 16384
_OBS = 256
_OUT = 64
_H = 512
_K = 10
_TEST = 100
_BT = 2048
_NT = _B // _BT
_LOG2PI = float(np.log(2.0 * np.pi))


def _memo_body(x_ref, dx_ref, a_ref,
               ve_w1, ve_b1, ve_w2, ve_b2, pre_w, pre_b,
               embT, emb, emb_sq, post_w, post_b,
               vd_w1, vd_b1, vd_w2, vd_b2,
               d_w1a, d_w1b, d_b1, d_w2, d_b2, d_w3, d_b3, d_w4, d_b4,
               ls_ref,
               reconl_ref, prop_ref, sr_ref, sq_ref):
    dx = dx_ref[...]
    h = jnp.tanh(jnp.dot(dx, ve_w1[...],
                         preferred_element_type=jnp.float32) + ve_b1[...])
    enc0 = jnp.dot(h, ve_w2[...],
                   preferred_element_type=jnp.float32) + ve_b2[...]
    enc = jnp.dot(enc0, pre_w[...],
                  preferred_element_type=jnp.float32) + pre_b[...]
    score = emb_sq[...] - 2.0 * jnp.dot(enc, embT[...],
                                        preferred_element_type=jnp.float32)
    mind = jnp.min(score, axis=1, keepdims=True)
    idxr = jax.lax.broadcasted_iota(jnp.int32, (_BT, _K), 1)
    prop = jnp.min(jnp.where(score == mind, idxr, _K), axis=1)
    onehot = (idxr == prop[:, None]).astype(jnp.float32)
    quant = jnp.dot(onehot, emb[...], preferred_element_type=jnp.float32)
    postq = jnp.dot(quant, post_w[...],
                    preferred_element_type=jnp.float32) + post_b[...]
    t1 = jnp.tanh(jnp.dot(postq, vd_w1[...],
                          preferred_element_type=jnp.float32) + vd_b1[...])
    recon = jnp.tanh(jnp.dot(t1, vd_w2[...],
                             preferred_element_type=jnp.float32) + vd_b2[...])
    propf = prop.astype(jnp.float32)
    h1 = jax.nn.relu(jnp.dot(x_ref[...], d_w1a[...],
                             preferred_element_type=jnp.float32)
                     + propf[:, None] * d_w1b[...] + d_b1[...])
    h2 = jax.nn.relu(jnp.dot(h1, d_w2[...],
                             preferred_element_type=jnp.float32) + d_b2[...])
    h3 = jnp.tanh(jax.nn.relu(jnp.dot(h2, d_w3[...],
                                      preferred_element_type=jnp.float32)
                              + d_b3[...]))
    mu = jnp.dot(h3, d_w4[...],
                 preferred_element_type=jnp.float32) + d_b4[...]
    ls = ls_ref[...]
    z = (a_ref[...] - mu) * jnp.exp(-ls)
    rl = jnp.sum(0.5 * z * z + ls + 0.5 * _LOG2PI, axis=1)
    reconl_ref[0, 0, :] = rl
    prop_ref[0, 0, :] = prop
    dr = dx - recon
    dq = enc - quant
    sr_ref[...] = jnp.full((1, 1, 128), jnp.sum(dr * dr), jnp.float32)
    sq_ref[...] = jnp.full((1, 1, 128), jnp.sum(dq * dq), jnp.float32)


def _final_body(reconl_ref, sr_ref, sq_ref, loss_ref, vqt_ref):
    tot = (jnp.sum(sr_ref[...]) + 2.0 * jnp.sum(sq_ref[...])) / 128.0
    vq_total = tot * (1.0 / (_B * _OBS))
    vqt_ref[...] = jnp.full((1, 128), vq_total, jnp.float32)
    loss_ref[...] = reconl_ref[...].reshape(_NT, _BT) * vq_total


def _tile_map(i):
    return (i, 0)


def _whole(i):
    return (0, 0)


def kernel(X, Delta_X, A, context_sample, con_dim, ve_w1, ve_b1, ve_w2, ve_b2,
           pre_w, pre_b, emb, post_w, post_b, vd_w1, vd_b1, vd_w2, vd_b2,
           d_w1, d_b1, d_w2, d_b2, d_w3, d_b3, d_w4, d_b4, log_std):
    embT = emb.T
    emb_sq = jnp.sum(emb * emb, axis=1)[None, :]
    d_w1a = d_w1[:_OBS]
    d_w1b = d_w1[_OBS:]

    def row(v):
        return v.reshape(1, -1)

    ins = (X, Delta_X, A,
           ve_w1, row(ve_b1), ve_w2, row(ve_b2), pre_w, row(pre_b),
           embT, emb, emb_sq, post_w, row(post_b),
           vd_w1, row(vd_b1), vd_w2, row(vd_b2),
           d_w1a, d_w1b, row(d_b1), d_w2, row(d_b2), d_w3, row(d_b3),
           d_w4, row(d_b4), row(log_std))

    in_specs = [
        pl.BlockSpec((_BT, _OBS), _tile_map),
        pl.BlockSpec((_BT, _OBS), _tile_map),
        pl.BlockSpec((_BT, _OUT), _tile_map),
    ] + [pl.BlockSpec(v.shape, _whole) for v in ins[3:]]

    recon2, prop3, sr, sq = pl.pallas_call(
        _memo_body,
        grid=(_NT,),
        in_specs=in_specs,
        out_specs=(
            pl.BlockSpec((1, 1, _BT), lambda i: (i, 0, 0)),
            pl.BlockSpec((1, 1, _BT), lambda i: (i, 0, 0)),
            pl.BlockSpec((1, 1, 128), lambda i: (i, 0, 0)),
            pl.BlockSpec((1, 1, 128), lambda i: (i, 0, 0)),
        ),
        out_shape=(
            jax.ShapeDtypeStruct((_NT, 1, _BT), jnp.float32),
            jax.ShapeDtypeStruct((_NT, 1, _BT), jnp.int32),
            jax.ShapeDtypeStruct((_NT, 1, 128), jnp.float32),
            jax.ShapeDtypeStruct((_NT, 1, 128), jnp.float32),
        ),
        compiler_params=pltpu.CompilerParams(
            dimension_semantics=("parallel",)),
    )(*ins)

    loss2, vqt = pl.pallas_call(
        _final_body,
        in_specs=[
            pl.BlockSpec((_NT, 1, _BT), lambda: (0, 0, 0)),
            pl.BlockSpec((_NT, 1, 128), lambda: (0, 0, 0)),
            pl.BlockSpec((_NT, 1, 128), lambda: (0, 0, 0)),
        ],
        out_specs=(
            pl.BlockSpec((_NT, _BT), lambda: (0, 0)),
            pl.BlockSpec((1, 128), lambda: (0, 0)),
        ),
        out_shape=(
            jax.ShapeDtypeStruct((_NT, _BT), jnp.float32),
            jax.ShapeDtypeStruct((1, 128), jnp.float32),
        ),
    )(recon2, sr, sq)

    return (loss2.reshape(_B), recon2.reshape(_B), X, prop3.reshape(_B),
            vqt[0, 0])
